# baseline (speedup 1.0000x reference)
import functools

import jax
import jax.numpy as jnp
from jax import lax
from jax.experimental import pallas as pl
from jax.experimental.pallas import tpu as pltpu
from jax.experimental.pallas import tpu_sc as plsc

LATENT = 32
HIDDEN = 64
NC, NS = 2, 16
NW = NC * NS


def _pad_rows(x, mult, fill=0):
    n = x.shape[0]
    pad = (-n) % mult
    if pad == 0:
        return x
    return jnp.concatenate(
        [x, jnp.full((pad,) + x.shape[1:], fill, x.dtype)], axis=0)


def _silu(x):
    return x * jax.nn.sigmoid(x)


def _bd4(W):
    return jnp.kron(jnp.eye(4, dtype=W.dtype), W)


def _sc_gather_multi(items):
    D = LATENT
    mesh = plsc.VectorSubcoreMesh(core_axis_name="c", subcore_axis_name="s")
    maxc = max(chunk for _, _, chunk in items)
    out_types = [jax.ShapeDtypeStruct((idx.shape[0], D), jnp.float32)
                 for _, idx, _ in items]
    plans = [(idx.shape[0] // NW, chunk) for _, idx, chunk in items]

    @functools.partial(
        pl.kernel, mesh=mesh,
        out_type=tuple(out_types),
        scratch_types=[pltpu.VMEM((maxc,), jnp.int32),
                       pltpu.VMEM((maxc,), jnp.int32),
                       pltpu.VMEM((maxc, D), jnp.float32),
                       pltpu.VMEM((maxc, D), jnp.float32),
                       pltpu.SemaphoreType.DMA,
                       pltpu.SemaphoreType.DMA],
        compiler_params=pltpu.CompilerParams(use_tc_tiling_on_sc=False),
    )
    def k(*refs):
        n = len(items)
        tables = refs[0:2 * n:2]
        idxs = refs[1:2 * n:2]
        outs = refs[2 * n:3 * n]
        ia, ib, ra, rb, sa, sb = refs[3 * n:3 * n + 6]
        wid = lax.axis_index("s") * NC + lax.axis_index("c")

        for it in range(n):
            b_per_w, chunk = plans[it]
            iters = b_per_w // chunk
            t_hbm = tables[it]
            i_hbm = idxs[it]
            o_hbm = outs[it]
            iv_a, iv_b = ia.at[pl.ds(0, chunk)], ib.at[pl.ds(0, chunk)]
            rv_a, rv_b = ra.at[pl.ds(0, chunk)], rb.at[pl.ds(0, chunk)]

            def one(i, iv, rv, sem):
                base = wid * b_per_w + i * chunk
                pltpu.sync_copy(i_hbm.at[pl.ds(base, chunk)], iv)
                h = pltpu.async_copy(t_hbm.at[iv], rv, sem)
                return base, h

            def flush(base, h, rv):
                h.wait()
                pltpu.sync_copy(rv, o_hbm.at[pl.ds(base, chunk)])

            def body(p, _):
                b0, h0 = one(2 * p, iv_a, rv_a, sa)
                b1, h1 = one(2 * p + 1, iv_b, rv_b, sb)
                flush(b0, h0, rv_a)
                flush(b1, h1, rv_b)
                return 0

            lax.fori_loop(0, iters // 2, body, 0)
            if iters % 2:
                b0, h0 = one(iters - 1, iv_a, rv_a, sa)
                flush(b0, h0, rv_a)

    outs = k(*[x for (t, idx, _) in items for x in (t, idx)])
    return list(outs) if isinstance(outs, (tuple, list)) else [outs]


def _sc_gather(table4, idx, *, chunk):
    return _sc_gather_multi([(table4, idx, chunk)])[0]


@functools.partial(jax.jit, static_argnames=("n_seg_pad", "chunk"))
def _sc_scatter_add(vals, idx, *, n_seg_pad, chunk):
    B, D = vals.shape
    b_per_w = B // NW
    iters = b_per_w // chunk
    z = n_seg_pad // NS
    mesh = plsc.VectorSubcoreMesh(core_axis_name="c", subcore_axis_name="s")
    zeros = jnp.zeros((n_seg_pad, D), jnp.float32)

    @functools.partial(
        pl.kernel, mesh=mesh,
        out_type=jax.ShapeDtypeStruct((NC, n_seg_pad, D), jnp.float32),
        scratch_types=[pltpu.VMEM((chunk,), jnp.int32),
                       pltpu.VMEM((chunk, D), jnp.float32),
                       pltpu.VMEM_SHARED((n_seg_pad, D), jnp.float32)],
        compiler_params=pltpu.CompilerParams(use_tc_tiling_on_sc=False),
    )
    def k(vals_hbm, idx_hbm, zeros_hbm, out_hbm, idx_v, rows_v, acc_sh):
        cid = lax.axis_index("c")
        sid = lax.axis_index("s")
        pltpu.sync_copy(zeros_hbm.at[pl.ds(sid * z, z)],
                        acc_sh.at[pl.ds(sid * z, z)])
        plsc.subcore_barrier()

        def body(i, _):
            base = (sid * NC + cid) * b_per_w + i * chunk
            pltpu.sync_copy(idx_hbm.at[pl.ds(base, chunk)], idx_v)
            pltpu.sync_copy(vals_hbm.at[pl.ds(base, chunk)], rows_v)
            pltpu.sync_copy(rows_v, acc_sh.at[idx_v], add=True)
            return 0

        lax.fori_loop(0, iters, body, 0)
        plsc.subcore_barrier()
        pltpu.sync_copy(acc_sh.at[pl.ds(sid * z, z)],
                        out_hbm.at[cid].at[pl.ds(sid * z, z)])

    return k(vals, idx, zeros)


def _mlp3_body(a_ref, b_ref, c_ref, w1a_ref, w1b_ref, w1c_ref, b1_ref,
               w2_ref, b2_ref, res_ref, o_ref, *, out_act):
    h = (jnp.dot(a_ref[...], w1a_ref[...], preferred_element_type=jnp.float32)
         + jnp.dot(b_ref[...], w1b_ref[...], preferred_element_type=jnp.float32)
         + jnp.dot(c_ref[...], w1c_ref[...], preferred_element_type=jnp.float32)
         + b1_ref[...])
    h = _silu(h)
    o = jnp.dot(h, w2_ref[...], preferred_element_type=jnp.float32) + b2_ref[...]
    if out_act == "silu":
        o = _silu(o)
    else:
        o = jnp.tanh(o)
    o_ref[...] = o + res_ref[...]


def _mlp3p_body(a_ref, bp_ref, cp_ref, w1a_ref, w1b_ref, w1c_ref, b1_ref,
                w2_ref, b2_ref, res_ref, o_ref, *, out_act):
    b = bp_ref[0] + bp_ref[1]
    c = cp_ref[0] + cp_ref[1]
    h = (jnp.dot(a_ref[...], w1a_ref[...], preferred_element_type=jnp.float32)
         + jnp.dot(b, w1b_ref[...], preferred_element_type=jnp.float32)
         + jnp.dot(c, w1c_ref[...], preferred_element_type=jnp.float32)
         + b1_ref[...])
    h = _silu(h)
    o = jnp.dot(h, w2_ref[...], preferred_element_type=jnp.float32) + b2_ref[...]
    if out_act == "silu":
        o = _silu(o)
    else:
        o = jnp.tanh(o)
    o_ref[...] = o + res_ref[...]


def _mlp3_packed(a4, b4, c4, W1, b1, W2, b2, res4, out_act, block_n4,
                 pairs=False):
    n4 = a4.shape[0]
    grid = (n4 + block_n4 - 1) // block_n4
    w1a = _bd4(W1[:LATENT])
    w1b = _bd4(W1[LATENT:2 * LATENT])
    w1c = _bd4(W1[2 * LATENT:])
    w2 = _bd4(W2)
    b1t = jnp.tile(b1, 4).reshape(1, 4 * HIDDEN)
    b2t = jnp.tile(b2, 4).reshape(1, 4 * LATENT)
    row = pl.BlockSpec((block_n4, 128), lambda i: (i, 0))
    bc = pl.BlockSpec((2, block_n4, 128), lambda i: (0, i, 0)) if pairs else row
    full = lambda s: pl.BlockSpec(s, lambda i: (0,) * len(s))
    body = _mlp3p_body if pairs else _mlp3_body
    return pl.pallas_call(
        functools.partial(body, out_act=out_act),
        grid=(grid,),
        in_specs=[row, bc, bc,
                  full((128, 4 * HIDDEN)), full((128, 4 * HIDDEN)),
                  full((128, 4 * HIDDEN)), full((1, 4 * HIDDEN)),
                  full((4 * HIDDEN, 128)), full((1, 128)), row],
        out_specs=row,
        out_shape=jax.ShapeDtypeStruct((n4, 128), jnp.float32),
    )(a4, b4, c4, w1a, w1b, w1c, b1t, w2, b2t, res4)


def _score_body(a_ref, b_ref, w1a_ref, w1b_ref, b1_ref, w2_ref, b2_ref,
                lg_ref, p_ref, o_ref):
    h = jnp.tanh(
        jnp.dot(a_ref[...], w1a_ref[...], preferred_element_type=jnp.float32)
        + jnp.dot(b_ref[...], w1b_ref[...], preferred_element_type=jnp.float32)
        + b1_ref[...])
    s = jnp.dot(h, w2_ref[...], preferred_element_type=jnp.float32) + b2_ref[...]
    att = jnp.exp(lg_ref[...] + s)
    o_ref[...] = jnp.dot(att, p_ref[...], preferred_element_type=jnp.float32)


def _bs_scores(a4, b4, W1, b1, W2, b2, lg4, P, block_n4):
    n4 = a4.shape[0]
    grid = (n4 + block_n4 - 1) // block_n4
    full = lambda s: pl.BlockSpec(s, lambda i: (0,) * len(s))
    row = pl.BlockSpec((block_n4, 128), lambda i: (i, 0))
    return pl.pallas_call(
        _score_body,
        grid=(grid,),
        in_specs=[row, row,
                  full((128, 4 * HIDDEN)), full((128, 4 * HIDDEN)),
                  full((1, 4 * HIDDEN)), full((4 * HIDDEN, 4)), full((1, 4)),
                  pl.BlockSpec((block_n4, 4), lambda i: (i, 0)),
                  full((4, 128))],
        out_specs=row,
        out_shape=jax.ShapeDtypeStruct((n4, 128), jnp.float32),
    )(a4, b4, _bd4(W1[:LATENT]), _bd4(W1[LATENT:]),
      jnp.tile(b1, 4).reshape(1, 4 * HIDDEN), _bd4(W2),
      jnp.tile(b2, 4).reshape(1, 4), lg4, P)


def _pair_add_body(p_ref, o_ref):
    o_ref[...] = p_ref[0] + p_ref[1]


def _pair_add(p):
    n4 = p.shape[1]
    return pl.pallas_call(
        _pair_add_body,
        grid=(1,),
        in_specs=[pl.BlockSpec((2, n4, 128), lambda i: (0, 0, 0))],
        out_specs=pl.BlockSpec((n4, 128), lambda i: (0, 0)),
        out_shape=jax.ShapeDtypeStruct((n4, 128), jnp.float32),
    )(p)


def _attn_body(am_ref, dr_ref, nb_ref, pt_ref, q_ref, attn_ref, vnm_ref):
    att0 = jnp.dot(am_ref[...], pt_ref[...],
                   preferred_element_type=jnp.float32)
    dn = jnp.dot(dr_ref[...], pt_ref[...],
                 preferred_element_type=jnp.float32)
    attn = att0 / (1e-12 + dn)
    attn_ref[...] = attn
    bcast = jnp.dot(attn, q_ref[...], preferred_element_type=jnp.float32)
    vnm_ref[...] = bcast * nb_ref[...]


def _attn_combine(attmat4, drows4, nb04, PT, Q, block_n4):
    n4 = attmat4.shape[0]
    grid = (n4 + block_n4 - 1) // block_n4
    row = pl.BlockSpec((block_n4, 128), lambda i: (i, 0))
    full = lambda s: pl.BlockSpec(s, lambda i: (0,) * len(s))
    return pl.pallas_call(
        _attn_body,
        grid=(grid,),
        in_specs=[row, row, row, full((128, 4)), full((4, 128))],
        out_specs=(pl.BlockSpec((block_n4, 4), lambda i: (i, 0)), row),
        out_shape=(jax.ShapeDtypeStruct((n4, 4), jnp.float32),
                   jax.ShapeDtypeStruct((n4, 128), jnp.float32)),
    )(attmat4, drows4, nb04, PT, Q)


def _rowscale_body(a_ref, s_ref, q_ref, o_ref):
    bcast = jnp.dot(s_ref[...], q_ref[...], preferred_element_type=jnp.float32)
    o_ref[...] = bcast * a_ref[...]


def _rowscale(a4, s4, Q, block_n4):
    n4 = a4.shape[0]
    grid = (n4 + block_n4 - 1) // block_n4
    row = pl.BlockSpec((block_n4, 128), lambda i: (i, 0))
    full = lambda s: pl.BlockSpec(s, lambda i: (0,) * len(s))
    return pl.pallas_call(
        _rowscale_body,
        grid=(grid,),
        in_specs=[row, pl.BlockSpec((block_n4, 4), lambda i: (i, 0)),
                  full((4, 128))],
        out_specs=row,
        out_shape=jax.ShapeDtypeStruct((n4, 128), jnp.float32),
    )(a4, s4, Q)


def kernel(nodes, edges, supernodes, superedges, graph, bipartite_graph,
           bipartite_graph_attention_logits, super_graph,
           super_graph_attention, en_W1, en_b1, en_W2, en_b2, nn_W1, nn_b1,
           nn_W2, nn_b2, sn_W1, sn_b1, sn_W2, sn_b2, se_W1, se_b1, se_W2,
           se_b2, bs_W1, bs_b1, bs_W2, bs_b2):
    g0, g1 = graph[0], graph[1]
    bg0, bg1 = bipartite_graph[0], bipartite_graph[1]
    sg0, sg1 = super_graph[0], super_graph[1]
    NPAD = 50048
    SPAD = 1024

    e4 = jnp.reshape(edges, (200000, 128))
    n4 = jnp.reshape(nodes, (12500, 128))
    sup_pad = _pad_rows(supernodes, 1024)
    s4 = jnp.reshape(sup_pad, (256, 128))
    se4 = jnp.reshape(superedges, (4000, 128))
    pk = lambda x: jnp.reshape(x, (x.shape[0] // 4, 128))
    pkp = lambda x: jnp.reshape(x, (2, x.shape[1] // 4, 128))
    unpk = lambda x: jnp.reshape(x, (x.shape[0] * 4, LATENT))

    eye4 = jnp.eye(4, dtype=jnp.float32)
    P = jnp.kron(eye4, jax.nn.one_hot(0, LATENT, dtype=jnp.float32)[None, :])
    PT = P.T
    Q = jnp.kron(eye4, jnp.ones((1, LATENT), jnp.float32))

    bg0g = _pad_rows(bg0, NW * 1600, 0)
    bg0s = _pad_rows(bg0, NW * 1600, NPAD - 1)
    bg1g = _pad_rows(bg1, NW * 1600, 0)
    bg1s = _pad_rows(bg1, NW * 1600, SPAD - 1)
    sg0g = _pad_rows(sg0, NW * 512, 0)
    sg1g = _pad_rows(sg1, NW * 512, 0)
    sg1s = _pad_rows(sg1, NW * 512, SPAD - 1)

    nb0, sb1, se0 = _sc_gather_multi([(nodes, bg0g, 1600),
                                      (sup_pad, bg1g, 1600),
                                      (superedges, sg0g, 512)])
    lg4 = jnp.reshape(_pad_rows(bipartite_graph_attention_logits, NW * 1600),
                      (25600, 4))
    attmat4 = _bs_scores(pk(nb0), pk(sb1), bs_W1, bs_b1, bs_W2, bs_b2,
                         lg4, P, 1600)
    dpair = _sc_scatter_add(unpk(attmat4), bg0s, n_seg_pad=NPAD, chunk=800)
    dmat4 = _pair_add(pkp(dpair))
    drows = _sc_gather(unpk(dmat4), bg0g, chunk=1600)
    attn4, vals_nm4 = _attn_combine(attmat4, pk(drows), pk(nb0), PT, Q, 1600)

    nm_pair = _sc_scatter_add(unpk(vals_nm4), bg1s, n_seg_pad=SPAD,
                              chunk=1600)
    sga4 = jnp.reshape(_pad_rows(super_graph_attention, NW * 512), (4096, 4))
    vals_am4 = _rowscale(pk(se0), sga4, Q, 512)
    am_pair = _sc_scatter_add(unpk(vals_am4), sg1s, n_seg_pad=SPAD, chunk=512)
    sup4 = _mlp3_packed(s4, pkp(am_pair), pkp(nm_pair), sn_W1, sn_b1, sn_W2,
                        sn_b2, s4, "silu", 256, pairs=True)
    supc = unpk(sup4)

    sup_b1, sup_s0, sup_s1 = _sc_gather_multi([(supc, bg1g, 1600),
                                               (supc, sg0g, 512),
                                               (supc, sg1g, 512)])
    vals_sm4 = _rowscale(pk(sup_b1), attn4, Q, 1600)
    sm_pair = _sc_scatter_add(unpk(vals_sm4), bg0s, n_seg_pad=NPAD, chunk=800)
    em_pair = _sc_scatter_add(unpk(e4), g1, n_seg_pad=NPAD, chunk=200)
    nod4 = _mlp3_packed(n4, pkp(em_pair), pkp(sm_pair), nn_W1, nn_b1, nn_W2,
                        nn_b2, n4, "silu", 1600, pairs=True)
    nodc = unpk(nod4)

    sed4 = _mlp3_packed(pk(sup_s0)[:4000], pk(sup_s1)[:4000], se4,
                        se_W1, se_b1, se_W2, se_b2, se4, "tanh", 1000)

    x0, x1 = _sc_gather_multi([(nodc, g0, 1000), (nodc, g1, 1000)])
    edg4 = _mlp3_packed(pk(x0), pk(x1), e4, en_W1, en_b1, en_W2, en_b2,
                        e4, "tanh", 2000)

    return (nodc,
            jnp.reshape(edg4, (800000, LATENT)),
            supc[:1000],
            jnp.reshape(sed4, (16000, LATENT)))


# speedup vs baseline: 1.0592x; 1.0592x over previous
import functools

import jax
import jax.numpy as jnp
from jax import lax
from jax.experimental import pallas as pl
from jax.experimental.pallas import tpu as pltpu
from jax.experimental.pallas import tpu_sc as plsc

LATENT = 32
HIDDEN = 64
NC, NS = 2, 16
NW = NC * NS


def _pad_rows(x, mult, fill=0):
    n = x.shape[0]
    pad = (-n) % mult
    if pad == 0:
        return x
    return jnp.concatenate(
        [x, jnp.full((pad,) + x.shape[1:], fill, x.dtype)], axis=0)


def _silu(x):
    return x * jax.nn.sigmoid(x)


def _bd4(W):
    return jnp.kron(jnp.eye(4, dtype=W.dtype), W)


def _sc_gather_multi(items):
    D = LATENT
    mesh = plsc.VectorSubcoreMesh(core_axis_name="c", subcore_axis_name="s")
    maxc = max(chunk for _, _, chunk in items)
    out_types = [jax.ShapeDtypeStruct((idx.shape[0], D), jnp.float32)
                 for _, idx, _ in items]
    plans = [(idx.shape[0] // NW, chunk) for _, idx, chunk in items]

    @functools.partial(
        pl.kernel, mesh=mesh,
        out_type=tuple(out_types),
        scratch_types=[pltpu.VMEM((maxc,), jnp.int32),
                       pltpu.VMEM((maxc,), jnp.int32),
                       pltpu.VMEM((maxc, D), jnp.float32),
                       pltpu.VMEM((maxc, D), jnp.float32),
                       pltpu.SemaphoreType.DMA,
                       pltpu.SemaphoreType.DMA],
        compiler_params=pltpu.CompilerParams(use_tc_tiling_on_sc=False),
    )
    def k(*refs):
        n = len(items)
        tables = refs[0:2 * n:2]
        idxs = refs[1:2 * n:2]
        outs = refs[2 * n:3 * n]
        ia, ib, ra, rb, sa, sb = refs[3 * n:3 * n + 6]
        wid = lax.axis_index("s") * NC + lax.axis_index("c")

        for it in range(n):
            b_per_w, chunk = plans[it]
            iters = b_per_w // chunk
            t_hbm = tables[it]
            i_hbm = idxs[it]
            o_hbm = outs[it]
            iv_a, iv_b = ia.at[pl.ds(0, chunk)], ib.at[pl.ds(0, chunk)]
            rv_a, rv_b = ra.at[pl.ds(0, chunk)], rb.at[pl.ds(0, chunk)]

            def one(i, iv, rv, sem):
                base = wid * b_per_w + i * chunk
                pltpu.sync_copy(i_hbm.at[pl.ds(base, chunk)], iv)
                h = pltpu.async_copy(t_hbm.at[iv], rv, sem)
                return base, h

            def flush(base, h, rv):
                h.wait()
                pltpu.sync_copy(rv, o_hbm.at[pl.ds(base, chunk)])

            def body(p, _):
                b0, h0 = one(2 * p, iv_a, rv_a, sa)
                b1, h1 = one(2 * p + 1, iv_b, rv_b, sb)
                flush(b0, h0, rv_a)
                flush(b1, h1, rv_b)
                return 0

            lax.fori_loop(0, iters // 2, body, 0)
            if iters % 2:
                b0, h0 = one(iters - 1, iv_a, rv_a, sa)
                flush(b0, h0, rv_a)

    outs = k(*[x for (t, idx, _) in items for x in (t, idx)])
    return list(outs) if isinstance(outs, (tuple, list)) else [outs]


def _sc_gather(table4, idx, *, chunk):
    return _sc_gather_multi([(table4, idx, chunk)])[0]


@functools.partial(jax.jit, static_argnames=("n_seg_pad", "chunk"))
def _sc_scatter_add(vals, idx, *, n_seg_pad, chunk):
    B, D = vals.shape
    b_per_w = B // NW
    iters = b_per_w // chunk
    z = n_seg_pad // NS
    mesh = plsc.VectorSubcoreMesh(core_axis_name="c", subcore_axis_name="s")
    zeros = jnp.zeros((n_seg_pad, D), jnp.float32)

    @functools.partial(
        pl.kernel, mesh=mesh,
        out_type=jax.ShapeDtypeStruct((NC, n_seg_pad, D), jnp.float32),
        scratch_types=[pltpu.VMEM((chunk,), jnp.int32),
                       pltpu.VMEM((chunk,), jnp.int32),
                       pltpu.VMEM((chunk, D), jnp.float32),
                       pltpu.VMEM((chunk, D), jnp.float32),
                       pltpu.VMEM_SHARED((n_seg_pad, D), jnp.float32),
                       pltpu.SemaphoreType.DMA, pltpu.SemaphoreType.DMA,
                       pltpu.SemaphoreType.DMA, pltpu.SemaphoreType.DMA],
        compiler_params=pltpu.CompilerParams(use_tc_tiling_on_sc=False),
    )
    def k(vals_hbm, idx_hbm, zeros_hbm, out_hbm, ia, ib, ra, rb, acc_sh,
          sla, slb, ssa, ssb):
        cid = lax.axis_index("c")
        sid = lax.axis_index("s")
        pltpu.sync_copy(zeros_hbm.at[pl.ds(sid * z, z)],
                        acc_sh.at[pl.ds(sid * z, z)])
        plsc.subcore_barrier()

        def load(i, iv, rv, sl):
            base = (sid * NC + cid) * b_per_w + i * chunk
            hi = pltpu.async_copy(idx_hbm.at[pl.ds(base, chunk)], iv, sl)
            hv = pltpu.async_copy(vals_hbm.at[pl.ds(base, chunk)], rv, sl)
            return hi, hv

        def scat(h, iv, rv, ss):
            h[0].wait()
            h[1].wait()
            return pltpu.async_copy(rv, acc_sh.at[iv], ss, add=True)

        def body(p, _):
            ha = load(2 * p, ia, ra, sla)
            hb = load(2 * p + 1, ib, rb, slb)
            wa = scat(ha, ia, ra, ssa)
            wb = scat(hb, ib, rb, ssb)
            wa.wait()
            wb.wait()
            return 0

        lax.fori_loop(0, iters // 2, body, 0)
        if iters % 2:
            ha = load(iters - 1, ia, ra, sla)
            scat(ha, ia, ra, ssa).wait()
        plsc.subcore_barrier()
        pltpu.sync_copy(acc_sh.at[pl.ds(sid * z, z)],
                        out_hbm.at[cid].at[pl.ds(sid * z, z)])

    return k(vals, idx, zeros)


def _mlp3_body(a_ref, b_ref, c_ref, w1a_ref, w1b_ref, w1c_ref, b1_ref,
               w2_ref, b2_ref, res_ref, o_ref, *, out_act):
    h = (jnp.dot(a_ref[...], w1a_ref[...], preferred_element_type=jnp.float32)
         + jnp.dot(b_ref[...], w1b_ref[...], preferred_element_type=jnp.float32)
         + jnp.dot(c_ref[...], w1c_ref[...], preferred_element_type=jnp.float32)
         + b1_ref[...])
    h = _silu(h)
    o = jnp.dot(h, w2_ref[...], preferred_element_type=jnp.float32) + b2_ref[...]
    if out_act == "silu":
        o = _silu(o)
    else:
        o = jnp.tanh(o)
    o_ref[...] = o + res_ref[...]


def _mlp3p_body(a_ref, bp_ref, cp_ref, w1a_ref, w1b_ref, w1c_ref, b1_ref,
                w2_ref, b2_ref, res_ref, o_ref, *, out_act):
    b = bp_ref[0] + bp_ref[1]
    c = cp_ref[0] + cp_ref[1]
    h = (jnp.dot(a_ref[...], w1a_ref[...], preferred_element_type=jnp.float32)
         + jnp.dot(b, w1b_ref[...], preferred_element_type=jnp.float32)
         + jnp.dot(c, w1c_ref[...], preferred_element_type=jnp.float32)
         + b1_ref[...])
    h = _silu(h)
    o = jnp.dot(h, w2_ref[...], preferred_element_type=jnp.float32) + b2_ref[...]
    if out_act == "silu":
        o = _silu(o)
    else:
        o = jnp.tanh(o)
    o_ref[...] = o + res_ref[...]


def _mlp3_packed(a4, b4, c4, W1, b1, W2, b2, res4, out_act, block_n4,
                 pairs=False):
    n4 = a4.shape[0]
    grid = (n4 + block_n4 - 1) // block_n4
    w1a = _bd4(W1[:LATENT])
    w1b = _bd4(W1[LATENT:2 * LATENT])
    w1c = _bd4(W1[2 * LATENT:])
    w2 = _bd4(W2)
    b1t = jnp.tile(b1, 4).reshape(1, 4 * HIDDEN)
    b2t = jnp.tile(b2, 4).reshape(1, 4 * LATENT)
    row = pl.BlockSpec((block_n4, 128), lambda i: (i, 0))
    bc = pl.BlockSpec((2, block_n4, 128), lambda i: (0, i, 0)) if pairs else row
    full = lambda s: pl.BlockSpec(s, lambda i: (0,) * len(s))
    body = _mlp3p_body if pairs else _mlp3_body
    return pl.pallas_call(
        functools.partial(body, out_act=out_act),
        grid=(grid,),
        in_specs=[row, bc, bc,
                  full((128, 4 * HIDDEN)), full((128, 4 * HIDDEN)),
                  full((128, 4 * HIDDEN)), full((1, 4 * HIDDEN)),
                  full((4 * HIDDEN, 128)), full((1, 128)), row],
        out_specs=row,
        out_shape=jax.ShapeDtypeStruct((n4, 128), jnp.float32),
    )(a4, b4, c4, w1a, w1b, w1c, b1t, w2, b2t, res4)


def _score_body(a_ref, b_ref, w1a_ref, w1b_ref, b1_ref, w2_ref, b2_ref,
                lg_ref, p_ref, o_ref):
    h = jnp.tanh(
        jnp.dot(a_ref[...], w1a_ref[...], preferred_element_type=jnp.float32)
        + jnp.dot(b_ref[...], w1b_ref[...], preferred_element_type=jnp.float32)
        + b1_ref[...])
    s = jnp.dot(h, w2_ref[...], preferred_element_type=jnp.float32) + b2_ref[...]
    att = jnp.exp(lg_ref[...] + s)
    o_ref[...] = jnp.dot(att, p_ref[...], preferred_element_type=jnp.float32)


def _bs_scores(a4, b4, W1, b1, W2, b2, lg4, P, block_n4):
    n4 = a4.shape[0]
    grid = (n4 + block_n4 - 1) // block_n4
    full = lambda s: pl.BlockSpec(s, lambda i: (0,) * len(s))
    row = pl.BlockSpec((block_n4, 128), lambda i: (i, 0))
    return pl.pallas_call(
        _score_body,
        grid=(grid,),
        in_specs=[row, row,
                  full((128, 4 * HIDDEN)), full((128, 4 * HIDDEN)),
                  full((1, 4 * HIDDEN)), full((4 * HIDDEN, 4)), full((1, 4)),
                  pl.BlockSpec((block_n4, 4), lambda i: (i, 0)),
                  full((4, 128))],
        out_specs=row,
        out_shape=jax.ShapeDtypeStruct((n4, 128), jnp.float32),
    )(a4, b4, _bd4(W1[:LATENT]), _bd4(W1[LATENT:]),
      jnp.tile(b1, 4).reshape(1, 4 * HIDDEN), _bd4(W2),
      jnp.tile(b2, 4).reshape(1, 4), lg4, P)


def _pair_add_body(p_ref, o_ref):
    o_ref[...] = p_ref[0] + p_ref[1]


def _pair_add(p):
    n4 = p.shape[1]
    return pl.pallas_call(
        _pair_add_body,
        grid=(1,),
        in_specs=[pl.BlockSpec((2, n4, 128), lambda i: (0, 0, 0))],
        out_specs=pl.BlockSpec((n4, 128), lambda i: (0, 0)),
        out_shape=jax.ShapeDtypeStruct((n4, 128), jnp.float32),
    )(p)


def _attn_body(am_ref, dr_ref, nb_ref, pt_ref, q_ref, attn_ref, vnm_ref):
    att0 = jnp.dot(am_ref[...], pt_ref[...],
                   preferred_element_type=jnp.float32)
    dn = jnp.dot(dr_ref[...], pt_ref[...],
                 preferred_element_type=jnp.float32)
    attn = att0 / (1e-12 + dn)
    attn_ref[...] = attn
    bcast = jnp.dot(attn, q_ref[...], preferred_element_type=jnp.float32)
    vnm_ref[...] = bcast * nb_ref[...]


def _attn_combine(attmat4, drows4, nb04, PT, Q, block_n4):
    n4 = attmat4.shape[0]
    grid = (n4 + block_n4 - 1) // block_n4
    row = pl.BlockSpec((block_n4, 128), lambda i: (i, 0))
    full = lambda s: pl.BlockSpec(s, lambda i: (0,) * len(s))
    return pl.pallas_call(
        _attn_body,
        grid=(grid,),
        in_specs=[row, row, row, full((128, 4)), full((4, 128))],
        out_specs=(pl.BlockSpec((block_n4, 4), lambda i: (i, 0)), row),
        out_shape=(jax.ShapeDtypeStruct((n4, 4), jnp.float32),
                   jax.ShapeDtypeStruct((n4, 128), jnp.float32)),
    )(attmat4, drows4, nb04, PT, Q)


def _rowscale_body(a_ref, s_ref, q_ref, o_ref):
    bcast = jnp.dot(s_ref[...], q_ref[...], preferred_element_type=jnp.float32)
    o_ref[...] = bcast * a_ref[...]


def _rowscale(a4, s4, Q, block_n4):
    n4 = a4.shape[0]
    grid = (n4 + block_n4 - 1) // block_n4
    row = pl.BlockSpec((block_n4, 128), lambda i: (i, 0))
    full = lambda s: pl.BlockSpec(s, lambda i: (0,) * len(s))
    return pl.pallas_call(
        _rowscale_body,
        grid=(grid,),
        in_specs=[row, pl.BlockSpec((block_n4, 4), lambda i: (i, 0)),
                  full((4, 128))],
        out_specs=row,
        out_shape=jax.ShapeDtypeStruct((n4, 128), jnp.float32),
    )(a4, s4, Q)


def kernel(nodes, edges, supernodes, superedges, graph, bipartite_graph,
           bipartite_graph_attention_logits, super_graph,
           super_graph_attention, en_W1, en_b1, en_W2, en_b2, nn_W1, nn_b1,
           nn_W2, nn_b2, sn_W1, sn_b1, sn_W2, sn_b2, se_W1, se_b1, se_W2,
           se_b2, bs_W1, bs_b1, bs_W2, bs_b2):
    g0, g1 = graph[0], graph[1]
    bg0, bg1 = bipartite_graph[0], bipartite_graph[1]
    sg0, sg1 = super_graph[0], super_graph[1]
    NPAD = 50048
    SPAD = 1024

    e4 = jnp.reshape(edges, (200000, 128))
    n4 = jnp.reshape(nodes, (12500, 128))
    sup_pad = _pad_rows(supernodes, 1024)
    s4 = jnp.reshape(sup_pad, (256, 128))
    se4 = jnp.reshape(superedges, (4000, 128))
    pk = lambda x: jnp.reshape(x, (x.shape[0] // 4, 128))
    pkp = lambda x: jnp.reshape(x, (2, x.shape[1] // 4, 128))
    unpk = lambda x: jnp.reshape(x, (x.shape[0] * 4, LATENT))

    eye4 = jnp.eye(4, dtype=jnp.float32)
    P = jnp.kron(eye4, jax.nn.one_hot(0, LATENT, dtype=jnp.float32)[None, :])
    PT = P.T
    Q = jnp.kron(eye4, jnp.ones((1, LATENT), jnp.float32))

    bg0g = _pad_rows(bg0, NW * 1600, 0)
    bg0s = _pad_rows(bg0, NW * 1600, NPAD - 1)
    bg1g = _pad_rows(bg1, NW * 1600, 0)
    bg1s = _pad_rows(bg1, NW * 1600, SPAD - 1)
    sg0g = _pad_rows(sg0, NW * 512, 0)
    sg1g = _pad_rows(sg1, NW * 512, 0)
    sg1s = _pad_rows(sg1, NW * 512, SPAD - 1)

    nb0, sb1, se0 = _sc_gather_multi([(nodes, bg0g, 1600),
                                      (sup_pad, bg1g, 1600),
                                      (superedges, sg0g, 512)])
    lg4 = jnp.reshape(_pad_rows(bipartite_graph_attention_logits, NW * 1600),
                      (25600, 4))
    attmat4 = _bs_scores(pk(nb0), pk(sb1), bs_W1, bs_b1, bs_W2, bs_b2,
                         lg4, P, 1600)
    dpair = _sc_scatter_add(unpk(attmat4), bg0s, n_seg_pad=NPAD, chunk=400)
    dmat4 = _pair_add(pkp(dpair))
    drows = _sc_gather(unpk(dmat4), bg0g, chunk=1600)
    attn4, vals_nm4 = _attn_combine(attmat4, pk(drows), pk(nb0), PT, Q, 1600)

    nm_pair = _sc_scatter_add(unpk(vals_nm4), bg1s, n_seg_pad=SPAD,
                              chunk=1600)
    sga4 = jnp.reshape(_pad_rows(super_graph_attention, NW * 512), (4096, 4))
    vals_am4 = _rowscale(pk(se0), sga4, Q, 512)
    am_pair = _sc_scatter_add(unpk(vals_am4), sg1s, n_seg_pad=SPAD, chunk=512)
    sup4 = _mlp3_packed(s4, pkp(am_pair), pkp(nm_pair), sn_W1, sn_b1, sn_W2,
                        sn_b2, s4, "silu", 256, pairs=True)
    supc = unpk(sup4)

    sup_b1, sup_s0, sup_s1 = _sc_gather_multi([(supc, bg1g, 1600),
                                               (supc, sg0g, 512),
                                               (supc, sg1g, 512)])
    vals_sm4 = _rowscale(pk(sup_b1), attn4, Q, 1600)
    sm_pair = _sc_scatter_add(unpk(vals_sm4), bg0s, n_seg_pad=NPAD, chunk=400)
    em_pair = _sc_scatter_add(unpk(e4), g1, n_seg_pad=NPAD, chunk=200)
    nod4 = _mlp3_packed(n4, pkp(em_pair), pkp(sm_pair), nn_W1, nn_b1, nn_W2,
                        nn_b2, n4, "silu", 1600, pairs=True)
    nodc = unpk(nod4)

    sed4 = _mlp3_packed(pk(sup_s0)[:4000], pk(sup_s1)[:4000], se4,
                        se_W1, se_b1, se_W2, se_b2, se4, "tanh", 1000)

    x0, x1 = _sc_gather_multi([(nodc, g0, 1000), (nodc, g1, 1000)])
    edg4 = _mlp3_packed(pk(x0), pk(x1), e4, en_W1, en_b1, en_W2, en_b2,
                        e4, "tanh", 2000)

    return (nodc,
            jnp.reshape(edg4, (800000, LATENT)),
            supc[:1000],
            jnp.reshape(sed4, (16000, LATENT)))


# speedup vs baseline: 1.0620x; 1.0026x over previous
import functools

import jax
import jax.numpy as jnp
from jax import lax
from jax.experimental import pallas as pl
from jax.experimental.pallas import tpu as pltpu
from jax.experimental.pallas import tpu_sc as plsc

LATENT = 32
HIDDEN = 64
NC, NS = 2, 16
NW = NC * NS


def _pad_rows(x, mult, fill=0):
    n = x.shape[0]
    pad = (-n) % mult
    if pad == 0:
        return x
    return jnp.concatenate(
        [x, jnp.full((pad,) + x.shape[1:], fill, x.dtype)], axis=0)


def _silu(x):
    return x * jax.nn.sigmoid(x)


def _bd4(W):
    return jnp.kron(jnp.eye(4, dtype=W.dtype), W)


def _sc_gather_multi(items):
    D = LATENT
    mesh = plsc.VectorSubcoreMesh(core_axis_name="c", subcore_axis_name="s")
    maxc = max(chunk for _, _, chunk in items)
    out_types = [jax.ShapeDtypeStruct((idx.shape[0], D), jnp.float32)
                 for _, idx, _ in items]
    plans = [(idx.shape[0] // NW, chunk) for _, idx, chunk in items]

    @functools.partial(
        pl.kernel, mesh=mesh,
        out_type=tuple(out_types),
        scratch_types=[pltpu.VMEM((maxc,), jnp.int32),
                       pltpu.VMEM((maxc,), jnp.int32),
                       pltpu.VMEM((maxc, D), jnp.float32),
                       pltpu.VMEM((maxc, D), jnp.float32),
                       pltpu.SemaphoreType.DMA,
                       pltpu.SemaphoreType.DMA],
        compiler_params=pltpu.CompilerParams(use_tc_tiling_on_sc=False),
    )
    def k(*refs):
        n = len(items)
        tables = refs[0:2 * n:2]
        idxs = refs[1:2 * n:2]
        outs = refs[2 * n:3 * n]
        ia, ib, ra, rb, sa, sb = refs[3 * n:3 * n + 6]
        wid = lax.axis_index("s") * NC + lax.axis_index("c")

        for it in range(n):
            b_per_w, chunk = plans[it]
            iters = b_per_w // chunk
            t_hbm = tables[it]
            i_hbm = idxs[it]
            o_hbm = outs[it]
            iv_a, iv_b = ia.at[pl.ds(0, chunk)], ib.at[pl.ds(0, chunk)]
            rv_a, rv_b = ra.at[pl.ds(0, chunk)], rb.at[pl.ds(0, chunk)]

            def one(i, iv, rv, sem):
                base = wid * b_per_w + i * chunk
                pltpu.sync_copy(i_hbm.at[pl.ds(base, chunk)], iv)
                h = pltpu.async_copy(t_hbm.at[iv], rv, sem)
                return base, h

            def flush(base, h, rv):
                h.wait()
                pltpu.sync_copy(rv, o_hbm.at[pl.ds(base, chunk)])

            def body(p, _):
                b0, h0 = one(2 * p, iv_a, rv_a, sa)
                b1, h1 = one(2 * p + 1, iv_b, rv_b, sb)
                flush(b0, h0, rv_a)
                flush(b1, h1, rv_b)
                return 0

            lax.fori_loop(0, iters // 2, body, 0)
            if iters % 2:
                b0, h0 = one(iters - 1, iv_a, rv_a, sa)
                flush(b0, h0, rv_a)

    outs = k(*[x for (t, idx, _) in items for x in (t, idx)])
    return list(outs) if isinstance(outs, (tuple, list)) else [outs]


def _sc_gather(table4, idx, *, chunk):
    return _sc_gather_multi([(table4, idx, chunk)])[0]


@functools.partial(jax.jit, static_argnames=("n_seg_pad", "chunk"))
def _sc_scatter_add(vals, idx, *, n_seg_pad, chunk):
    B, D = vals.shape
    b_per_w = B // NW
    iters = b_per_w // chunk
    z = n_seg_pad // NS
    mesh = plsc.VectorSubcoreMesh(core_axis_name="c", subcore_axis_name="s")
    zeros = jnp.zeros((n_seg_pad, D), jnp.float32)

    @functools.partial(
        pl.kernel, mesh=mesh,
        out_type=jax.ShapeDtypeStruct((NC, n_seg_pad, D), jnp.float32),
        scratch_types=[pltpu.VMEM((chunk,), jnp.int32),
                       pltpu.VMEM((chunk,), jnp.int32),
                       pltpu.VMEM((chunk, D), jnp.float32),
                       pltpu.VMEM((chunk, D), jnp.float32),
                       pltpu.VMEM_SHARED((n_seg_pad, D), jnp.float32),
                       pltpu.SemaphoreType.DMA, pltpu.SemaphoreType.DMA,
                       pltpu.SemaphoreType.DMA, pltpu.SemaphoreType.DMA],
        compiler_params=pltpu.CompilerParams(use_tc_tiling_on_sc=False),
    )
    def k(vals_hbm, idx_hbm, zeros_hbm, out_hbm, ia, ib, ra, rb, acc_sh,
          sla, slb, ssa, ssb):
        cid = lax.axis_index("c")
        sid = lax.axis_index("s")
        pltpu.sync_copy(zeros_hbm.at[pl.ds(sid * z, z)],
                        acc_sh.at[pl.ds(sid * z, z)])
        plsc.subcore_barrier()

        def load(i, iv, rv, sl):
            base = (sid * NC + cid) * b_per_w + i * chunk
            hi = pltpu.async_copy(idx_hbm.at[pl.ds(base, chunk)], iv, sl)
            hv = pltpu.async_copy(vals_hbm.at[pl.ds(base, chunk)], rv, sl)
            return hi, hv

        def scat(h, iv, rv, ss):
            h[0].wait()
            h[1].wait()
            return pltpu.async_copy(rv, acc_sh.at[iv], ss, add=True)

        def body(p, _):
            ha = load(2 * p, ia, ra, sla)
            hb = load(2 * p + 1, ib, rb, slb)
            wa = scat(ha, ia, ra, ssa)
            wb = scat(hb, ib, rb, ssb)
            wa.wait()
            wb.wait()
            return 0

        lax.fori_loop(0, iters // 2, body, 0)
        if iters % 2:
            ha = load(iters - 1, ia, ra, sla)
            scat(ha, ia, ra, ssa).wait()
        plsc.subcore_barrier()
        pltpu.sync_copy(acc_sh.at[pl.ds(sid * z, z)],
                        out_hbm.at[cid].at[pl.ds(sid * z, z)])

    return k(vals, idx, zeros)


def _dot(x, w):
    return jnp.dot(x.astype(jnp.bfloat16), w.astype(jnp.bfloat16),
                   preferred_element_type=jnp.float32)


def _mlp3_body(a_ref, b_ref, c_ref, w1a_ref, w1b_ref, w1c_ref, b1_ref,
               w2_ref, b2_ref, res_ref, o_ref, *, out_act):
    h = (_dot(a_ref[...], w1a_ref[...]) + _dot(b_ref[...], w1b_ref[...])
         + _dot(c_ref[...], w1c_ref[...]) + b1_ref[...])
    h = _silu(h)
    o = _dot(h, w2_ref[...]) + b2_ref[...]
    if out_act == "silu":
        o = _silu(o)
    else:
        o = jnp.tanh(o)
    o_ref[...] = o + res_ref[...]


def _mlp3p_body(a_ref, bp_ref, cp_ref, w1a_ref, w1b_ref, w1c_ref, b1_ref,
                w2_ref, b2_ref, res_ref, o_ref, *, out_act):
    b = bp_ref[0] + bp_ref[1]
    c = cp_ref[0] + cp_ref[1]
    h = (_dot(a_ref[...], w1a_ref[...]) + _dot(b, w1b_ref[...])
         + _dot(c, w1c_ref[...]) + b1_ref[...])
    h = _silu(h)
    o = _dot(h, w2_ref[...]) + b2_ref[...]
    if out_act == "silu":
        o = _silu(o)
    else:
        o = jnp.tanh(o)
    o_ref[...] = o + res_ref[...]


def _mlp3_packed(a4, b4, c4, W1, b1, W2, b2, res4, out_act, block_n4,
                 pairs=False):
    n4 = a4.shape[0]
    grid = (n4 + block_n4 - 1) // block_n4
    w1a = _bd4(W1[:LATENT])
    w1b = _bd4(W1[LATENT:2 * LATENT])
    w1c = _bd4(W1[2 * LATENT:])
    w2 = _bd4(W2)
    b1t = jnp.tile(b1, 4).reshape(1, 4 * HIDDEN)
    b2t = jnp.tile(b2, 4).reshape(1, 4 * LATENT)
    row = pl.BlockSpec((block_n4, 128), lambda i: (i, 0))
    bc = pl.BlockSpec((2, block_n4, 128), lambda i: (0, i, 0)) if pairs else row
    full = lambda s: pl.BlockSpec(s, lambda i: (0,) * len(s))
    body = _mlp3p_body if pairs else _mlp3_body
    return pl.pallas_call(
        functools.partial(body, out_act=out_act),
        grid=(grid,),
        in_specs=[row, bc, bc,
                  full((128, 4 * HIDDEN)), full((128, 4 * HIDDEN)),
                  full((128, 4 * HIDDEN)), full((1, 4 * HIDDEN)),
                  full((4 * HIDDEN, 128)), full((1, 128)), row],
        out_specs=row,
        out_shape=jax.ShapeDtypeStruct((n4, 128), jnp.float32),
    )(a4, b4, c4, w1a, w1b, w1c, b1t, w2, b2t, res4)


def _score_body(a_ref, b_ref, w1a_ref, w1b_ref, b1_ref, w2_ref, b2_ref,
                lg_ref, p_ref, o_ref):
    h = jnp.tanh(
        jnp.dot(a_ref[...], w1a_ref[...], preferred_element_type=jnp.float32)
        + jnp.dot(b_ref[...], w1b_ref[...], preferred_element_type=jnp.float32)
        + b1_ref[...])
    s = jnp.dot(h, w2_ref[...], preferred_element_type=jnp.float32) + b2_ref[...]
    att = jnp.exp(lg_ref[...] + s)
    o_ref[...] = jnp.dot(att, p_ref[...], preferred_element_type=jnp.float32)


def _bs_scores(a4, b4, W1, b1, W2, b2, lg4, P, block_n4):
    n4 = a4.shape[0]
    grid = (n4 + block_n4 - 1) // block_n4
    full = lambda s: pl.BlockSpec(s, lambda i: (0,) * len(s))
    row = pl.BlockSpec((block_n4, 128), lambda i: (i, 0))
    return pl.pallas_call(
        _score_body,
        grid=(grid,),
        in_specs=[row, row,
                  full((128, 4 * HIDDEN)), full((128, 4 * HIDDEN)),
                  full((1, 4 * HIDDEN)), full((4 * HIDDEN, 4)), full((1, 4)),
                  pl.BlockSpec((block_n4, 4), lambda i: (i, 0)),
                  full((4, 128))],
        out_specs=row,
        out_shape=jax.ShapeDtypeStruct((n4, 128), jnp.float32),
    )(a4, b4, _bd4(W1[:LATENT]), _bd4(W1[LATENT:]),
      jnp.tile(b1, 4).reshape(1, 4 * HIDDEN), _bd4(W2),
      jnp.tile(b2, 4).reshape(1, 4), lg4, P)


def _pair_add_body(p_ref, o_ref):
    o_ref[...] = p_ref[0] + p_ref[1]


def _pair_add(p):
    n4 = p.shape[1]
    return pl.pallas_call(
        _pair_add_body,
        grid=(1,),
        in_specs=[pl.BlockSpec((2, n4, 128), lambda i: (0, 0, 0))],
        out_specs=pl.BlockSpec((n4, 128), lambda i: (0, 0)),
        out_shape=jax.ShapeDtypeStruct((n4, 128), jnp.float32),
    )(p)


def _attn_body(am_ref, dr_ref, nb_ref, pt_ref, q_ref, attn_ref, vnm_ref):
    att0 = jnp.dot(am_ref[...], pt_ref[...],
                   preferred_element_type=jnp.float32)
    dn = jnp.dot(dr_ref[...], pt_ref[...],
                 preferred_element_type=jnp.float32)
    attn = att0 / (1e-12 + dn)
    attn_ref[...] = attn
    bcast = jnp.dot(attn, q_ref[...], preferred_element_type=jnp.float32)
    vnm_ref[...] = bcast * nb_ref[...]


def _attn_combine(attmat4, drows4, nb04, PT, Q, block_n4):
    n4 = attmat4.shape[0]
    grid = (n4 + block_n4 - 1) // block_n4
    row = pl.BlockSpec((block_n4, 128), lambda i: (i, 0))
    full = lambda s: pl.BlockSpec(s, lambda i: (0,) * len(s))
    return pl.pallas_call(
        _attn_body,
        grid=(grid,),
        in_specs=[row, row, row, full((128, 4)), full((4, 128))],
        out_specs=(pl.BlockSpec((block_n4, 4), lambda i: (i, 0)), row),
        out_shape=(jax.ShapeDtypeStruct((n4, 4), jnp.float32),
                   jax.ShapeDtypeStruct((n4, 128), jnp.float32)),
    )(attmat4, drows4, nb04, PT, Q)


def _rowscale_body(a_ref, s_ref, q_ref, o_ref):
    bcast = jnp.dot(s_ref[...], q_ref[...], preferred_element_type=jnp.float32)
    o_ref[...] = bcast * a_ref[...]


def _rowscale(a4, s4, Q, block_n4):
    n4 = a4.shape[0]
    grid = (n4 + block_n4 - 1) // block_n4
    row = pl.BlockSpec((block_n4, 128), lambda i: (i, 0))
    full = lambda s: pl.BlockSpec(s, lambda i: (0,) * len(s))
    return pl.pallas_call(
        _rowscale_body,
        grid=(grid,),
        in_specs=[row, pl.BlockSpec((block_n4, 4), lambda i: (i, 0)),
                  full((4, 128))],
        out_specs=row,
        out_shape=jax.ShapeDtypeStruct((n4, 128), jnp.float32),
    )(a4, s4, Q)


def kernel(nodes, edges, supernodes, superedges, graph, bipartite_graph,
           bipartite_graph_attention_logits, super_graph,
           super_graph_attention, en_W1, en_b1, en_W2, en_b2, nn_W1, nn_b1,
           nn_W2, nn_b2, sn_W1, sn_b1, sn_W2, sn_b2, se_W1, se_b1, se_W2,
           se_b2, bs_W1, bs_b1, bs_W2, bs_b2):
    g0, g1 = graph[0], graph[1]
    bg0, bg1 = bipartite_graph[0], bipartite_graph[1]
    sg0, sg1 = super_graph[0], super_graph[1]
    NPAD = 50048
    SPAD = 1024

    e4 = jnp.reshape(edges, (200000, 128))
    n4 = jnp.reshape(nodes, (12500, 128))
    sup_pad = _pad_rows(supernodes, 1024)
    s4 = jnp.reshape(sup_pad, (256, 128))
    se4 = jnp.reshape(superedges, (4000, 128))
    pk = lambda x: jnp.reshape(x, (x.shape[0] // 4, 128))
    pkp = lambda x: jnp.reshape(x, (2, x.shape[1] // 4, 128))
    unpk = lambda x: jnp.reshape(x, (x.shape[0] * 4, LATENT))

    eye4 = jnp.eye(4, dtype=jnp.float32)
    P = jnp.kron(eye4, jax.nn.one_hot(0, LATENT, dtype=jnp.float32)[None, :])
    PT = P.T
    Q = jnp.kron(eye4, jnp.ones((1, LATENT), jnp.float32))

    bg0g = _pad_rows(bg0, NW * 1600, 0)
    bg0s = _pad_rows(bg0, NW * 1600, NPAD - 1)
    bg1g = _pad_rows(bg1, NW * 1600, 0)
    bg1s = _pad_rows(bg1, NW * 1600, SPAD - 1)
    sg0g = _pad_rows(sg0, NW * 512, 0)
    sg1g = _pad_rows(sg1, NW * 512, 0)
    sg1s = _pad_rows(sg1, NW * 512, SPAD - 1)

    nb0, sb1, se0 = _sc_gather_multi([(nodes, bg0g, 1600),
                                      (sup_pad, bg1g, 1600),
                                      (superedges, sg0g, 512)])
    lg4 = jnp.reshape(_pad_rows(bipartite_graph_attention_logits, NW * 1600),
                      (25600, 4))
    attmat4 = _bs_scores(pk(nb0), pk(sb1), bs_W1, bs_b1, bs_W2, bs_b2,
                         lg4, P, 1600)
    dpair = _sc_scatter_add(unpk(attmat4), bg0s, n_seg_pad=NPAD, chunk=400)
    dmat4 = _pair_add(pkp(dpair))
    drows = _sc_gather(unpk(dmat4), bg0g, chunk=1600)
    attn4, vals_nm4 = _attn_combine(attmat4, pk(drows), pk(nb0), PT, Q, 1600)

    nm_pair = _sc_scatter_add(unpk(vals_nm4), bg1s, n_seg_pad=SPAD,
                              chunk=1600)
    sga4 = jnp.reshape(_pad_rows(super_graph_attention, NW * 512), (4096, 4))
    vals_am4 = _rowscale(pk(se0), sga4, Q, 512)
    am_pair = _sc_scatter_add(unpk(vals_am4), sg1s, n_seg_pad=SPAD, chunk=512)
    sup4 = _mlp3_packed(s4, pkp(am_pair), pkp(nm_pair), sn_W1, sn_b1, sn_W2,
                        sn_b2, s4, "silu", 256, pairs=True)
    supc = unpk(sup4)

    sup_b1, sup_s0, sup_s1 = _sc_gather_multi([(supc, bg1g, 1600),
                                               (supc, sg0g, 512),
                                               (supc, sg1g, 512)])
    vals_sm4 = _rowscale(pk(sup_b1), attn4, Q, 1600)
    sm_pair = _sc_scatter_add(unpk(vals_sm4), bg0s, n_seg_pad=NPAD, chunk=400)
    em_pair = _sc_scatter_add(unpk(e4), g1, n_seg_pad=NPAD, chunk=200)
    nod4 = _mlp3_packed(n4, pkp(em_pair), pkp(sm_pair), nn_W1, nn_b1, nn_W2,
                        nn_b2, n4, "silu", 1600, pairs=True)
    nodc = unpk(nod4)

    sed4 = _mlp3_packed(pk(sup_s0)[:4000], pk(sup_s1)[:4000], se4,
                        se_W1, se_b1, se_W2, se_b2, se4, "tanh", 1000)

    x0, x1 = _sc_gather_multi([(nodc, g0, 1000), (nodc, g1, 1000)])
    edg4 = _mlp3_packed(pk(x0), pk(x1), e4, en_W1, en_b1, en_W2, en_b2,
                        e4, "tanh", 2000)

    return (nodc,
            jnp.reshape(edg4, (800000, LATENT)),
            supc[:1000],
            jnp.reshape(sed4, (16000, LATENT)))
